# baseline (speedup 1.0000x reference)
import jax
import jax.numpy as jnp
from jax import lax
from jax.experimental import pallas as pl
from jax.experimental.pallas import tpu as pltpu
from jax.experimental.pallas import tpu_sc as plsc

N = 10000
D = 128
E = 320000

NP = 10016
RB = 1000
NBLK = N // RB
DE = 144
DE_AL = 129
NT = 32
BATCH = 96
NBATCH = 108
CH = 2
EPT = NBATCH * BATCH
EPAD = NT * EPT
RPT = NP // 16


def _emit_g(g_ref, alad_ref, g, as_ref, ad_ref):
    g_ref[:, pl.ds(0, 128)] = g
    al = jnp.sum(g * as_ref[...], axis=1)
    lane = lax.broadcasted_iota(jnp.int32, (RB, 16), 1)
    g_ref[:, pl.ds(128, 16)] = (jnp.where(lane == 0, 1.0, 0.0)
                                + jnp.where(lane == 1, al[:, None], 0.0))
    alad_ref[0, 0, :] = al
    alad_ref[0, 1, :] = jnp.sum(g * ad_ref[...], axis=1)


def _enc_body(x_ref, encW_ref, encb_ref, W0_ref, as_ref, ad_ref,
              h_ref, g_ref, alad_ref):
    h = jnp.dot(x_ref[...], encW_ref[...],
                preferred_element_type=jnp.float32) + encb_ref[...]
    h_ref[...] = h
    g = jnp.dot(h, W0_ref[...], preferred_element_type=jnp.float32)
    _emit_g(g_ref, alad_ref, g, as_ref, ad_ref)


def _post_layer(h2p_ref, hin_ref, bi_ref, lnw_ref, lnb_ref):
    num = h2p_ref[0, :, pl.ds(0, 128)] + h2p_ref[1, :, pl.ds(0, 128)]
    den = h2p_ref[0, :, pl.ds(128, 1)] + h2p_ref[1, :, pl.ds(128, 1)]
    h2 = num / (den + 1e-16) + bi_ref[...]
    mu = jnp.mean(h2, axis=1, keepdims=True)
    zc = h2 - mu
    var = jnp.mean(zc * zc, axis=1, keepdims=True)
    h2n = zc / jnp.sqrt(var + 1e-5) * lnw_ref[...] + lnb_ref[...]
    return jnp.maximum(h2n, 0.0) + hin_ref[...]


def _mid_body(h2p_ref, hin_ref, bi_ref, lnw_ref, lnb_ref, Wn_ref, as_ref,
              ad_ref, hout_ref, g_ref, alad_ref):
    hout = _post_layer(h2p_ref, hin_ref, bi_ref, lnw_ref, lnb_ref)
    hout_ref[...] = hout
    g = jnp.dot(hout, Wn_ref[...], preferred_element_type=jnp.float32)
    _emit_g(g_ref, alad_ref, g, as_ref, ad_ref)


def _fin_body(h2p_ref, hin_ref, bi_ref, lnw_ref, lnb_ref, decW_ref, decb_ref,
              out_ref):
    hout = _post_layer(h2p_ref, hin_ref, bi_ref, lnw_ref, lnb_ref)
    logits = jnp.dot(hout, decW_ref[...],
                     preferred_element_type=jnp.float32) + decb_ref[...]
    sg = jax.nn.sigmoid(logits)

    @pl.when(pl.program_id(0) == 0)
    def _():
        out_ref[...] = jnp.zeros_like(out_ref)

    out_ref[...] += jnp.sum(sg, axis=0, keepdims=True)


_full = lambda shape: pl.BlockSpec(shape, lambda i: tuple(0 for _ in shape))

_enc_call = pl.pallas_call(
    _enc_body,
    grid=(NBLK,),
    in_specs=[
        pl.BlockSpec((RB, D), lambda i: (i, 0)),
        _full((D, D)), _full((1, D)), _full((D, D)), _full((1, D)),
        _full((1, D)),
    ],
    out_specs=[
        pl.BlockSpec((RB, D), lambda i: (i, 0)),
        pl.BlockSpec((RB, DE), lambda i: (i, 0)),
        pl.BlockSpec((1, 2, RB), lambda i: (i, 0, 0)),
    ],
    out_shape=[
        jax.ShapeDtypeStruct((N, D), jnp.float32),
        jax.ShapeDtypeStruct((NP, DE), jnp.float32),
        jax.ShapeDtypeStruct((NBLK, 2, RB), jnp.float32),
    ],
)

_mid_call = pl.pallas_call(
    _mid_body,
    grid=(NBLK,),
    in_specs=[
        pl.BlockSpec((2, RB, DE), lambda i: (0, i, 0)),
        pl.BlockSpec((RB, D), lambda i: (i, 0)),
        _full((1, D)), _full((1, D)), _full((1, D)), _full((D, D)),
        _full((1, D)), _full((1, D)),
    ],
    out_specs=[
        pl.BlockSpec((RB, D), lambda i: (i, 0)),
        pl.BlockSpec((RB, DE), lambda i: (i, 0)),
        pl.BlockSpec((1, 2, RB), lambda i: (i, 0, 0)),
    ],
    out_shape=[
        jax.ShapeDtypeStruct((N, D), jnp.float32),
        jax.ShapeDtypeStruct((NP, DE), jnp.float32),
        jax.ShapeDtypeStruct((NBLK, 2, RB), jnp.float32),
    ],
)

_fin_call = pl.pallas_call(
    _fin_body,
    grid=(NBLK,),
    in_specs=[
        pl.BlockSpec((2, RB, DE), lambda i: (0, i, 0)),
        pl.BlockSpec((RB, D), lambda i: (i, 0)),
        _full((1, D)), _full((1, D)), _full((1, D)), _full((D, D)),
        _full((1, D)),
    ],
    out_specs=pl.BlockSpec((1, D), lambda i: (0, 0)),
    out_shape=jax.ShapeDtypeStruct((1, D), jnp.float32),
)


def _sc_body(g_hbm, alad_hbm, src_hbm, dst_hbm, h2p_hbm,
             ad_v, src_c, dst_c, ex_v, rows_v, h2_sh, gsem, ssem):
    c = lax.axis_index("c")
    s = lax.axis_index("s")
    wid = s * 2 + c
    row0 = s * RPT

    zv = jnp.zeros((16,), jnp.float32)

    def _z(i, carry):
        for rb in range(2):
            for k in range(DE // 16):
                rows_v[rb, i, pl.ds(k * 16, 16)] = zv
        return carry

    lax.fori_loop(0, BATCH, _z, 0)
    for k in range(RPT // BATCH):
        pltpu.sync_copy(rows_v.at[0], h2_sh.at[pl.ds(row0 + k * BATCH, BATCH)])
    rem = RPT % BATCH
    pltpu.sync_copy(rows_v.at[0, pl.ds(0, rem)],
                    h2_sh.at[pl.ds(row0 + RPT - rem, rem)])

    for k in range(NBLK):
        pltpu.sync_copy(alad_hbm.at[k, 1], ad_v.at[pl.ds(k * RB, RB)])
    ad_v[pl.ds(N, NP - N)] = jnp.full((16,), -1e30, jnp.float32)
    plsc.subcore_barrier()

    pltpu.sync_copy(src_hbm.at[wid, pl.ds(0, CH)], src_c.at[0])
    pltpu.sync_copy(dst_hbm.at[wid, pl.ds(0, CH)], dst_c.at[0])
    pltpu.async_copy(g_hbm.at[src_c.at[0, 0]], rows_v.at[0], gsem)

    col1 = jnp.full((16,), DE_AL, jnp.int32)
    lanes = lax.iota(jnp.int32, 16)

    def _batch(b, carry):
        rb = lax.rem(b, 2)
        nrb = lax.rem(b + 1, 2)
        lbi = lax.rem(b, CH)
        qb = lax.rem(lax.div(b, CH), 2)
        nlbi = lax.rem(b + 1, CH)
        nqb = lax.rem(lax.div(b + 1, CH), 2)

        @pl.when(b >= 1)
        def _():
            pltpu.make_async_copy(rows_v.at[nrb],
                                  h2_sh.at[dst_c.at[0, 0]], ssem).wait()

        @pl.when((b + 1 < NBATCH) & (nlbi == 0))
        def _():
            pltpu.sync_copy(src_hbm.at[wid, pl.ds(b + 1, CH)], src_c.at[nqb])
            pltpu.sync_copy(dst_hbm.at[wid, pl.ds(b + 1, CH)], dst_c.at[nqb])

        @pl.when(b + 1 < NBATCH)
        def _():
            pltpu.async_copy(g_hbm.at[src_c.at[nqb, nlbi]], rows_v.at[nrb],
                             gsem)

        for j in range(BATCH // 16):
            dv = dst_c[qb, lbi, pl.ds(j * 16, 16)]
            ex_v[pl.ds(j * 16, 16)] = plsc.load_gather(ad_v, [dv])

        pltpu.make_async_copy(g_hbm.at[src_c.at[qb, lbi]], rows_v.at[rb],
                              gsem).wait()

        rbv = jnp.zeros((16,), jnp.int32) + rb

        def _scale(g16, inner):
            rids = g16 * 16 + lanes
            alg = plsc.load_gather(rows_v, [rbv, rids, col1])
            t = alg + ex_v[pl.ds(g16 * 16, 16)]
            ex = jnp.exp(jnp.maximum(t, 0.2 * t))
            for kk in range(16):
                sc = ex[kk]
                row = g16 * 16 + kk
                for k in range(DE // 16):
                    sl = pl.ds(k * 16, 16)
                    rows_v[rb, row, sl] = rows_v[rb, row, sl] * sc
            return inner

        lax.fori_loop(0, BATCH // 16, _scale, 0)
        pltpu.async_copy(rows_v.at[rb], h2_sh.at[dst_c.at[qb, lbi]], ssem,
                         add=True)
        return carry

    lax.fori_loop(0, NBATCH, _batch, 0)
    pltpu.make_async_copy(rows_v.at[0], h2_sh.at[dst_c.at[0, 0]], ssem).wait()

    plsc.subcore_barrier()
    pltpu.sync_copy(h2_sh.at[pl.ds(row0, RPT)],
                    h2p_hbm.at[c, pl.ds(row0, RPT)])


_sc_edge = pl.kernel(
    _sc_body,
    out_type=jax.ShapeDtypeStruct((2, NP, DE), jnp.float32),
    mesh=plsc.VectorSubcoreMesh(core_axis_name="c", subcore_axis_name="s"),
    scratch_types=[
        pltpu.VMEM((NP,), jnp.float32),
        pltpu.VMEM((2, CH, BATCH), jnp.int32),
        pltpu.VMEM((2, CH, BATCH), jnp.int32),
        pltpu.VMEM((BATCH,), jnp.float32),
        pltpu.VMEM((2, BATCH, DE), jnp.float32),
        pltpu.VMEM_SHARED((NP, DE), jnp.float32),
        pltpu.SemaphoreType.DMA,
        pltpu.SemaphoreType.DMA,
    ],
    compiler_params=pltpu.CompilerParams(needs_layout_passes=False,
                                         use_tc_tiling_on_sc=False),
)


def _impl(x, edge_index, batch, enc_W, enc_b, W, a_src, a_dst, b, ln_w, ln_b,
          dec_W, dec_b):
    pad = N + (jnp.arange(EPAD - E - N, dtype=jnp.int32) % (NP - N))
    loops = jnp.arange(N, dtype=jnp.int32)
    src = jnp.concatenate([edge_index[0].astype(jnp.int32), loops, pad])
    dst = jnp.concatenate([edge_index[1].astype(jnp.int32), loops, pad])
    src = src.reshape(NT, NBATCH, BATCH)
    dst = dst.reshape(NT, NBATCH, BATCH)

    r1 = lambda v: v.reshape(1, D)

    h0, g0, alad0 = _enc_call(x, enc_W, r1(enc_b), W[0], r1(a_src[0]),
                              r1(a_dst[0]))
    h2p0 = _sc_edge(g0, alad0, src, dst)
    h1, g1, alad1 = _mid_call(h2p0, h0, r1(b[0]), r1(ln_w[0]), r1(ln_b[0]),
                              W[1], r1(a_src[1]), r1(a_dst[1]))
    h2p1 = _sc_edge(g1, alad1, src, dst)
    out = _fin_call(h2p1, h1, r1(b[1]), r1(ln_w[1]), r1(ln_b[1]), dec_W,
                    r1(dec_b))
    return out.reshape(D)


kernel = jax.jit(_impl)


# speedup vs baseline: 2.1663x; 2.1663x over previous
import jax
import jax.numpy as jnp
from jax import lax
from jax.experimental import pallas as pl
from jax.experimental.pallas import tpu as pltpu
from jax.experimental.pallas import tpu_sc as plsc

N = 10000
D = 128
E = 320000

NP = 10016
RB = 1000
NBLK = N // RB
DE = 144
DE_AL = 129
NT = 32
BATCH = 96
NBATCH = 108
CH = 2
EPT = NBATCH * BATCH
EPAD = NT * EPT
RPT = NP // 16


def _emit_g(g_ref, alad_ref, g, as_ref, ad_ref):
    g_ref[:, pl.ds(0, 128)] = g
    al = jnp.sum(g * as_ref[...], axis=1)
    lane = lax.broadcasted_iota(jnp.int32, (RB, 16), 1)
    g_ref[:, pl.ds(128, 16)] = (jnp.where(lane == 0, 1.0, 0.0)
                                + jnp.where(lane == 1, al[:, None], 0.0))
    alad_ref[0, 0, :] = al
    alad_ref[0, 1, :] = jnp.sum(g * ad_ref[...], axis=1)


def _enc_body(x_ref, encW_ref, encb_ref, W0_ref, as_ref, ad_ref,
              h_ref, g_ref, alad_ref):
    h = jnp.dot(x_ref[...], encW_ref[...],
                preferred_element_type=jnp.float32) + encb_ref[...]
    h_ref[...] = h
    g = jnp.dot(h, W0_ref[...], preferred_element_type=jnp.float32)
    _emit_g(g_ref, alad_ref, g, as_ref, ad_ref)


def _post_layer(h2p_ref, hin_ref, bi_ref, lnw_ref, lnb_ref):
    num = h2p_ref[0, :, pl.ds(0, 128)] + h2p_ref[1, :, pl.ds(0, 128)]
    den = h2p_ref[0, :, pl.ds(128, 1)] + h2p_ref[1, :, pl.ds(128, 1)]
    h2 = num / (den + 1e-16) + bi_ref[...]
    mu = jnp.mean(h2, axis=1, keepdims=True)
    zc = h2 - mu
    var = jnp.mean(zc * zc, axis=1, keepdims=True)
    h2n = zc / jnp.sqrt(var + 1e-5) * lnw_ref[...] + lnb_ref[...]
    return jnp.maximum(h2n, 0.0) + hin_ref[...]


def _mid_body(h2p_ref, hin_ref, bi_ref, lnw_ref, lnb_ref, Wn_ref, as_ref,
              ad_ref, hout_ref, g_ref, alad_ref):
    hout = _post_layer(h2p_ref, hin_ref, bi_ref, lnw_ref, lnb_ref)
    hout_ref[...] = hout
    g = jnp.dot(hout, Wn_ref[...], preferred_element_type=jnp.float32)
    _emit_g(g_ref, alad_ref, g, as_ref, ad_ref)


def _fin_body(h2p_ref, hin_ref, bi_ref, lnw_ref, lnb_ref, decW_ref, decb_ref,
              out_ref):
    hout = _post_layer(h2p_ref, hin_ref, bi_ref, lnw_ref, lnb_ref)
    logits = jnp.dot(hout, decW_ref[...],
                     preferred_element_type=jnp.float32) + decb_ref[...]
    sg = jax.nn.sigmoid(logits)

    @pl.when(pl.program_id(0) == 0)
    def _():
        out_ref[...] = jnp.zeros_like(out_ref)

    out_ref[...] += jnp.sum(sg, axis=0, keepdims=True)


_full = lambda shape: pl.BlockSpec(shape, lambda i: tuple(0 for _ in shape))

_enc_call = pl.pallas_call(
    _enc_body,
    grid=(NBLK,),
    in_specs=[
        pl.BlockSpec((RB, D), lambda i: (i, 0)),
        _full((D, D)), _full((1, D)), _full((D, D)), _full((1, D)),
        _full((1, D)),
    ],
    out_specs=[
        pl.BlockSpec((RB, D), lambda i: (i, 0)),
        pl.BlockSpec((RB, DE), lambda i: (i, 0)),
        pl.BlockSpec((1, 2, RB), lambda i: (i, 0, 0)),
    ],
    out_shape=[
        jax.ShapeDtypeStruct((N, D), jnp.float32),
        jax.ShapeDtypeStruct((NP, DE), jnp.float32),
        jax.ShapeDtypeStruct((NBLK, 2, RB), jnp.float32),
    ],
)

_mid_call = pl.pallas_call(
    _mid_body,
    grid=(NBLK,),
    in_specs=[
        pl.BlockSpec((2, RB, DE), lambda i: (0, i, 0)),
        pl.BlockSpec((RB, D), lambda i: (i, 0)),
        _full((1, D)), _full((1, D)), _full((1, D)), _full((D, D)),
        _full((1, D)), _full((1, D)),
    ],
    out_specs=[
        pl.BlockSpec((RB, D), lambda i: (i, 0)),
        pl.BlockSpec((RB, DE), lambda i: (i, 0)),
        pl.BlockSpec((1, 2, RB), lambda i: (i, 0, 0)),
    ],
    out_shape=[
        jax.ShapeDtypeStruct((N, D), jnp.float32),
        jax.ShapeDtypeStruct((NP, DE), jnp.float32),
        jax.ShapeDtypeStruct((NBLK, 2, RB), jnp.float32),
    ],
)

_fin_call = pl.pallas_call(
    _fin_body,
    grid=(NBLK,),
    in_specs=[
        pl.BlockSpec((2, RB, DE), lambda i: (0, i, 0)),
        pl.BlockSpec((RB, D), lambda i: (i, 0)),
        _full((1, D)), _full((1, D)), _full((1, D)), _full((D, D)),
        _full((1, D)),
    ],
    out_specs=pl.BlockSpec((1, D), lambda i: (0, 0)),
    out_shape=jax.ShapeDtypeStruct((1, D), jnp.float32),
)


def _sc_body(g_hbm, alad_hbm, src_hbm, dst_hbm, h2p_hbm,
             ad_v, src_c, dst_c, ex_v, rows_v, h2_sh, gsem, ssem):
    c = lax.axis_index("c")
    s = lax.axis_index("s")
    wid = s * 2 + c
    row0 = s * RPT

    zv = jnp.zeros((16,), jnp.float32)

    def _z(i, carry):
        for rb in range(2):
            for k in range(DE // 16):
                rows_v[rb, i, pl.ds(k * 16, 16)] = zv
        return carry

    lax.fori_loop(0, BATCH, _z, 0)
    for k in range(RPT // BATCH):
        pltpu.sync_copy(rows_v.at[0], h2_sh.at[pl.ds(row0 + k * BATCH, BATCH)])
    rem = RPT % BATCH
    pltpu.sync_copy(rows_v.at[0, pl.ds(0, rem)],
                    h2_sh.at[pl.ds(row0 + RPT - rem, rem)])

    for k in range(NBLK):
        pltpu.sync_copy(alad_hbm.at[k, 1], ad_v.at[pl.ds(k * RB, RB)])
    ad_v[pl.ds(N, NP - N)] = jnp.full((16,), -1e30, jnp.float32)
    plsc.subcore_barrier()

    pltpu.sync_copy(src_hbm.at[wid, pl.ds(0, CH)], src_c.at[0])
    pltpu.sync_copy(dst_hbm.at[wid, pl.ds(0, CH)], dst_c.at[0])
    pltpu.async_copy(g_hbm.at[src_c.at[0, 0]], rows_v.at[0], gsem)

    col1 = jnp.full((16,), DE_AL, jnp.int32)
    lanes = lax.iota(jnp.int32, 16)

    def _batch(b, carry):
        rb = lax.rem(b, 2)
        nrb = lax.rem(b + 1, 2)
        lbi = lax.rem(b, CH)
        qb = lax.rem(lax.div(b, CH), 2)
        nlbi = lax.rem(b + 1, CH)
        nqb = lax.rem(lax.div(b + 1, CH), 2)

        @pl.when(b >= 1)
        def _():
            pltpu.make_async_copy(rows_v.at[nrb],
                                  h2_sh.at[dst_c.at[0, 0]], ssem).wait()

        @pl.when((b + 1 < NBATCH) & (nlbi == 0))
        def _():
            pltpu.sync_copy(src_hbm.at[wid, pl.ds(b + 1, CH)], src_c.at[nqb])
            pltpu.sync_copy(dst_hbm.at[wid, pl.ds(b + 1, CH)], dst_c.at[nqb])

        @pl.when(b + 1 < NBATCH)
        def _():
            pltpu.async_copy(g_hbm.at[src_c.at[nqb, nlbi]], rows_v.at[nrb],
                             gsem)

        for j in range(BATCH // 16):
            dv = dst_c[qb, lbi, pl.ds(j * 16, 16)]
            ex_v[pl.ds(j * 16, 16)] = plsc.load_gather(ad_v, [dv])

        def _work(rbuf):
            pltpu.make_async_copy(g_hbm.at[src_c.at[qb, lbi]], rbuf,
                                  gsem).wait()

            def _scale(g16, inner):
                rids = g16 * 16 + lanes
                alg = plsc.load_gather(rbuf, [rids, col1])
                t = alg + ex_v[pl.ds(g16 * 16, 16)]
                ex = jnp.exp(jnp.maximum(t, 0.2 * t))
                for kk in range(16):
                    sc = ex[kk]
                    row = g16 * 16 + kk
                    for k in range(DE // 16):
                        sl = pl.ds(k * 16, 16)
                        rbuf[row, sl] = rbuf[row, sl] * sc
                return inner

            lax.fori_loop(0, BATCH // 16, _scale, 0)
            pltpu.async_copy(rbuf, h2_sh.at[dst_c.at[qb, lbi]], ssem,
                             add=True)

        @pl.when(rb == 0)
        def _():
            _work(rows_v.at[0])

        @pl.when(rb == 1)
        def _():
            _work(rows_v.at[1])

        return carry

    lax.fori_loop(0, NBATCH, _batch, 0)
    pltpu.make_async_copy(rows_v.at[0], h2_sh.at[dst_c.at[0, 0]], ssem).wait()

    plsc.subcore_barrier()
    pltpu.sync_copy(h2_sh.at[pl.ds(row0, RPT)],
                    h2p_hbm.at[c, pl.ds(row0, RPT)])


_sc_edge = pl.kernel(
    _sc_body,
    out_type=jax.ShapeDtypeStruct((2, NP, DE), jnp.float32),
    mesh=plsc.VectorSubcoreMesh(core_axis_name="c", subcore_axis_name="s"),
    scratch_types=[
        pltpu.VMEM((NP,), jnp.float32),
        pltpu.VMEM((2, CH, BATCH), jnp.int32),
        pltpu.VMEM((2, CH, BATCH), jnp.int32),
        pltpu.VMEM((BATCH,), jnp.float32),
        pltpu.VMEM((2, BATCH, DE), jnp.float32),
        pltpu.VMEM_SHARED((NP, DE), jnp.float32),
        pltpu.SemaphoreType.DMA,
        pltpu.SemaphoreType.DMA,
    ],
    compiler_params=pltpu.CompilerParams(needs_layout_passes=False,
                                         use_tc_tiling_on_sc=False),
)


def _impl(x, edge_index, batch, enc_W, enc_b, W, a_src, a_dst, b, ln_w, ln_b,
          dec_W, dec_b):
    pad = N + (jnp.arange(EPAD - E - N, dtype=jnp.int32) % (NP - N))
    loops = jnp.arange(N, dtype=jnp.int32)
    src = jnp.concatenate([edge_index[0].astype(jnp.int32), loops, pad])
    dst = jnp.concatenate([edge_index[1].astype(jnp.int32), loops, pad])
    src = src.reshape(NT, NBATCH, BATCH)
    dst = dst.reshape(NT, NBATCH, BATCH)

    r1 = lambda v: v.reshape(1, D)

    h0, g0, alad0 = _enc_call(x, enc_W, r1(enc_b), W[0], r1(a_src[0]),
                              r1(a_dst[0]))
    h2p0 = _sc_edge(g0, alad0, src, dst)
    h1, g1, alad1 = _mid_call(h2p0, h0, r1(b[0]), r1(ln_w[0]), r1(ln_b[0]),
                              W[1], r1(a_src[1]), r1(a_dst[1]))
    h2p1 = _sc_edge(g1, alad1, src, dst)
    out = _fin_call(h2p1, h1, r1(b[1]), r1(ln_w[1]), r1(ln_b[1]), dec_W,
                    r1(dec_b))
    return out.reshape(D)


kernel = jax.jit(_impl)


# speedup vs baseline: 2.5309x; 1.1684x over previous
import jax
import jax.numpy as jnp
from jax import lax
from jax.experimental import pallas as pl
from jax.experimental.pallas import tpu as pltpu
from jax.experimental.pallas import tpu_sc as plsc

N = 10000
D = 128
E = 320000

NP = 10240
RB = 1024
NBLK = NP // RB
NT = 32
BATCH = 96
NBATCH = 108
CH = 2
EPT = NBATCH * BATCH
EPAD = NT * EPT
RPT = NP // 16
DB = NP // 128


def _emit_g(g_ref, alad_ref, g, as_ref, ad_ref):
    g_ref[...] = g
    alad_ref[0, 0, :] = jnp.sum(g * as_ref[...], axis=1)
    alad_ref[0, 1, :] = jnp.sum(g * ad_ref[...], axis=1)


def _enc_body(x_ref, encW_ref, encb_ref, W0_ref, as_ref, ad_ref,
              h_ref, g_ref, alad_ref):
    h = jnp.dot(x_ref[...], encW_ref[...],
                preferred_element_type=jnp.float32) + encb_ref[...]
    h_ref[...] = h
    g = jnp.dot(h, W0_ref[...], preferred_element_type=jnp.float32)
    _emit_g(g_ref, alad_ref, g, as_ref, ad_ref)


def _post_layer(h2p_ref, den_ref, hin_ref, bi_ref, lnw_ref, lnb_ref):
    num = h2p_ref[0] + h2p_ref[1]
    h2 = num / (den_ref[...] + 1e-16) + bi_ref[...]
    mu = jnp.mean(h2, axis=1, keepdims=True)
    zc = h2 - mu
    var = jnp.mean(zc * zc, axis=1, keepdims=True)
    h2n = zc / jnp.sqrt(var + 1e-5) * lnw_ref[...] + lnb_ref[...]
    return jnp.maximum(h2n, 0.0) + hin_ref[...]


def _mid_body(h2p_ref, den_ref, hin_ref, bi_ref, lnw_ref, lnb_ref, Wn_ref,
              as_ref, ad_ref, hout_ref, g_ref, alad_ref):
    hout = _post_layer(h2p_ref, den_ref, hin_ref, bi_ref, lnw_ref, lnb_ref)
    hout_ref[...] = hout
    g = jnp.dot(hout, Wn_ref[...], preferred_element_type=jnp.float32)
    _emit_g(g_ref, alad_ref, g, as_ref, ad_ref)


def _fin_body(h2p_ref, den_ref, hin_ref, bi_ref, lnw_ref, lnb_ref, decW_ref,
              decb_ref, out_ref):
    hout = _post_layer(h2p_ref, den_ref, hin_ref, bi_ref, lnw_ref, lnb_ref)
    logits = jnp.dot(hout, decW_ref[...],
                     preferred_element_type=jnp.float32) + decb_ref[...]
    sg = jax.nn.sigmoid(logits)
    rid = lax.broadcasted_iota(jnp.int32, (RB, 1), 0) + pl.program_id(0) * RB
    sg = jnp.where(rid < N, sg, 0.0)

    @pl.when(pl.program_id(0) == 0)
    def _():
        out_ref[...] = jnp.zeros_like(out_ref)

    out_ref[...] += jnp.sum(sg, axis=0, keepdims=True)


_full = lambda shape: pl.BlockSpec(shape, lambda i: tuple(0 for _ in shape))

_enc_call = pl.pallas_call(
    _enc_body,
    grid=(NBLK,),
    in_specs=[
        pl.BlockSpec((RB, D), lambda i: (i, 0)),
        _full((D, D)), _full((1, D)), _full((D, D)), _full((1, D)),
        _full((1, D)),
    ],
    out_specs=[
        pl.BlockSpec((RB, D), lambda i: (i, 0)),
        pl.BlockSpec((RB, D), lambda i: (i, 0)),
        pl.BlockSpec((1, 2, RB), lambda i: (i, 0, 0)),
    ],
    out_shape=[
        jax.ShapeDtypeStruct((NP, D), jnp.float32),
        jax.ShapeDtypeStruct((NP, D), jnp.float32),
        jax.ShapeDtypeStruct((NBLK, 2, RB), jnp.float32),
    ],
)

_mid_call = pl.pallas_call(
    _mid_body,
    grid=(NBLK,),
    in_specs=[
        pl.BlockSpec((2, RB, D), lambda i: (0, i, 0)),
        pl.BlockSpec((RB, 1), lambda i: (i, 0)),
        pl.BlockSpec((RB, D), lambda i: (i, 0)),
        _full((1, D)), _full((1, D)), _full((1, D)), _full((D, D)),
        _full((1, D)), _full((1, D)),
    ],
    out_specs=[
        pl.BlockSpec((RB, D), lambda i: (i, 0)),
        pl.BlockSpec((RB, D), lambda i: (i, 0)),
        pl.BlockSpec((1, 2, RB), lambda i: (i, 0, 0)),
    ],
    out_shape=[
        jax.ShapeDtypeStruct((NP, D), jnp.float32),
        jax.ShapeDtypeStruct((NP, D), jnp.float32),
        jax.ShapeDtypeStruct((NBLK, 2, RB), jnp.float32),
    ],
)

_fin_call = pl.pallas_call(
    _fin_body,
    grid=(NBLK,),
    in_specs=[
        pl.BlockSpec((2, RB, D), lambda i: (0, i, 0)),
        pl.BlockSpec((RB, 1), lambda i: (i, 0)),
        pl.BlockSpec((RB, D), lambda i: (i, 0)),
        _full((1, D)), _full((1, D)), _full((1, D)), _full((D, D)),
        _full((1, D)),
    ],
    out_specs=pl.BlockSpec((1, D), lambda i: (0, 0)),
    out_shape=jax.ShapeDtypeStruct((1, D), jnp.float32),
)


def _sc_body(g_hbm, alad_hbm, src_hbm, dst_hbm, h2p_hbm, den_hbm,
             al_v, ad_v, src_c, dst_c, ex_c, vbuf, rows_v, h2_sh, den_sh,
             gsem, ssem):
    c = lax.axis_index("c")
    s = lax.axis_index("s")
    wid = s * 2 + c
    row0 = s * RPT

    zv = jnp.zeros((16,), jnp.float32)

    def _z(i, carry):
        for k in range(D // 16):
            rows_v[0, i, pl.ds(k * 16, 16)] = zv
        return carry

    lax.fori_loop(0, BATCH, _z, 0)
    for k in range(RPT // BATCH):
        pltpu.sync_copy(rows_v.at[0],
                        h2_sh.at[pl.ds(row0 + k * BATCH, BATCH)])
    rem = RPT % BATCH
    pltpu.sync_copy(rows_v.at[0, pl.ds(0, rem)],
                    h2_sh.at[pl.ds(row0 + RPT - rem, rem)])
    for k in range(D // 16):
        vbuf[pl.ds(k * 16, 16)] = zv
    for k in range(RPT // D):
        pltpu.sync_copy(vbuf, den_sh.at[pl.ds(row0 + k * D, D)])

    for k in range(NBLK):
        pltpu.sync_copy(alad_hbm.at[k, 0], al_v.at[pl.ds(k * RB, RB)])
        pltpu.sync_copy(alad_hbm.at[k, 1], ad_v.at[pl.ds(k * RB, RB)])
    neg = jnp.full((16,), -1e30, jnp.float32)
    for k in range((NP - N) // 16):
        al_v[pl.ds(N + k * 16, 16)] = neg
        ad_v[pl.ds(N + k * 16, 16)] = neg
    plsc.subcore_barrier()

    pltpu.sync_copy(src_hbm.at[wid, pl.ds(0, CH)], src_c.at[0])
    pltpu.sync_copy(dst_hbm.at[wid, pl.ds(0, CH)], dst_c.at[0])
    pltpu.async_copy(g_hbm.at[src_c.at[0, 0]], rows_v.at[0], gsem)

    def _batch(b, carry):
        rb = lax.rem(b, 2)
        nrb = lax.rem(b + 1, 2)
        lbi = lax.rem(b, CH)
        qb = lax.rem(lax.div(b, CH), 2)
        nlbi = lax.rem(b + 1, CH)
        nqb = lax.rem(lax.div(b + 1, CH), 2)

        @pl.when(b >= 1)
        def _():
            pltpu.make_async_copy(rows_v.at[nrb],
                                  h2_sh.at[dst_c.at[0, 0]], ssem).wait()
            pltpu.make_async_copy(ex_c.at[nrb],
                                  den_sh.at[dst_c.at[0, 0]], ssem).wait()

        @pl.when((b + 1 < NBATCH) & (nlbi == 0))
        def _():
            pltpu.sync_copy(src_hbm.at[wid, pl.ds(b + 1, CH)], src_c.at[nqb])
            pltpu.sync_copy(dst_hbm.at[wid, pl.ds(b + 1, CH)], dst_c.at[nqb])

        @pl.when(b + 1 < NBATCH)
        def _():
            pltpu.async_copy(g_hbm.at[src_c.at[nqb, nlbi]], rows_v.at[nrb],
                             gsem)

        def _work(exbuf, rbuf):
            for j in range(BATCH // 16):
                sv = src_c[qb, lbi, pl.ds(j * 16, 16)]
                dv = dst_c[qb, lbi, pl.ds(j * 16, 16)]
                t = (plsc.load_gather(al_v, [sv])
                     + plsc.load_gather(ad_v, [dv]))
                exbuf[pl.ds(j * 16, 16)] = jnp.exp(jnp.maximum(t, 0.2 * t))

            pltpu.make_async_copy(g_hbm.at[src_c.at[qb, lbi]], rbuf,
                                  gsem).wait()

            def _scale(g16, inner):
                ex = exbuf[pl.ds(g16 * 16, 16)]
                for kk in range(16):
                    sc = ex[kk]
                    row = g16 * 16 + kk
                    for k in range(D // 16):
                        sl = pl.ds(k * 16, 16)
                        rbuf[row, sl] = rbuf[row, sl] * sc
                return inner

            lax.fori_loop(0, BATCH // 16, _scale, 0)
            pltpu.async_copy(rbuf, h2_sh.at[dst_c.at[qb, lbi]], ssem,
                             add=True)
            pltpu.async_copy(exbuf, den_sh.at[dst_c.at[qb, lbi]], ssem,
                             add=True)

        @pl.when(rb == 0)
        def _():
            _work(ex_c.at[0], rows_v.at[0])

        @pl.when(rb == 1)
        def _():
            _work(ex_c.at[1], rows_v.at[1])

        return carry

    lax.fori_loop(0, NBATCH, _batch, 0)
    pltpu.make_async_copy(rows_v.at[0], h2_sh.at[dst_c.at[0, 0]], ssem).wait()
    pltpu.make_async_copy(ex_c.at[0], den_sh.at[dst_c.at[0, 0]], ssem).wait()

    plsc.subcore_barrier()
    pltpu.sync_copy(h2_sh.at[pl.ds(row0, RPT)],
                    h2p_hbm.at[c, pl.ds(row0, RPT)])
    for k in range(RPT // D):
        pltpu.sync_copy(den_sh.at[pl.ds(row0 + k * D, D)], vbuf)
        pltpu.sync_copy(vbuf, den_hbm.at[c, s * (RPT // D) + k])


_sc_edge = pl.kernel(
    _sc_body,
    out_type=[
        jax.ShapeDtypeStruct((2, NP, D), jnp.float32),
        jax.ShapeDtypeStruct((2, DB, D), jnp.float32),
    ],
    mesh=plsc.VectorSubcoreMesh(core_axis_name="c", subcore_axis_name="s"),
    scratch_types=[
        pltpu.VMEM((NP,), jnp.float32),
        pltpu.VMEM((NP,), jnp.float32),
        pltpu.VMEM((2, CH, BATCH), jnp.int32),
        pltpu.VMEM((2, CH, BATCH), jnp.int32),
        pltpu.VMEM((2, BATCH), jnp.float32),
        pltpu.VMEM((D,), jnp.float32),
        pltpu.VMEM((2, BATCH, D), jnp.float32),
        pltpu.VMEM_SHARED((NP, D), jnp.float32),
        pltpu.VMEM_SHARED((NP,), jnp.float32),
        pltpu.SemaphoreType.DMA,
        pltpu.SemaphoreType.DMA,
    ],
    compiler_params=pltpu.CompilerParams(needs_layout_passes=False,
                                         use_tc_tiling_on_sc=False),
)


def _impl(x, edge_index, batch, enc_W, enc_b, W, a_src, a_dst, b, ln_w, ln_b,
          dec_W, dec_b):
    pad = N + (jnp.arange(EPAD - E - N, dtype=jnp.int32) % (NP - N))
    loops = jnp.arange(N, dtype=jnp.int32)
    src = jnp.concatenate([edge_index[0].astype(jnp.int32), loops, pad])
    dst = jnp.concatenate([edge_index[1].astype(jnp.int32), loops, pad])
    src = src.reshape(NT, NBATCH, BATCH)
    dst = dst.reshape(NT, NBATCH, BATCH)

    xp = jnp.zeros((NP, D), jnp.float32).at[:N].set(x)
    r1 = lambda v: v.reshape(1, D)

    h0, g0, alad0 = _enc_call(xp, enc_W, r1(enc_b), W[0], r1(a_src[0]),
                              r1(a_dst[0]))
    h2p0, den0 = _sc_edge(g0, alad0, src, dst)
    dn0 = (den0[0] + den0[1]).reshape(NP, 1)
    h1, g1, alad1 = _mid_call(h2p0, dn0, h0, r1(b[0]), r1(ln_w[0]),
                              r1(ln_b[0]), W[1], r1(a_src[1]), r1(a_dst[1]))
    h2p1, den1 = _sc_edge(g1, alad1, src, dst)
    dn1 = (den1[0] + den1[1]).reshape(NP, 1)
    out = _fin_call(h2p1, dn1, h1, r1(b[1]), r1(ln_w[1]), r1(ln_b[1]),
                    dec_W, r1(dec_b))
    return out.reshape(D)


kernel = jax.jit(_impl)


# speedup vs baseline: 2.9498x; 1.1655x over previous
import jax
import jax.numpy as jnp
from jax import lax
from jax.experimental import pallas as pl
from jax.experimental.pallas import tpu as pltpu
from jax.experimental.pallas import tpu_sc as plsc

N = 10000
D = 128
E = 320000

NP = 10240
RB = 1024
NBLK = NP // RB
NT = 32
BATCH = 96
NBATCH = 108
CH = 2
EPT = NBATCH * BATCH
EPAD = NT * EPT
RPT = NP // 16
DB = NP // 128


def _emit_g(g_ref, alad_ref, g, as_ref, ad_ref):
    g_ref[...] = g
    al = jnp.sum(g * as_ref[...], axis=1)
    ad = jnp.sum(g * ad_ref[...], axis=1)
    ai = lax.bitcast_convert_type(al, jnp.int32) + 0x8000
    di = lax.bitcast_convert_type(ad, jnp.int32) + 0x8000
    alad_ref[0, 0, :] = (ai & jnp.int32(-65536)) | lax.shift_right_logical(
        di, 16)


def _enc_body(x_ref, encW_ref, encb_ref, W0_ref, as_ref, ad_ref,
              h_ref, g_ref, alad_ref):
    h = jnp.dot(x_ref[...], encW_ref[...],
                preferred_element_type=jnp.float32) + encb_ref[...]
    h_ref[...] = h
    g = jnp.dot(h, W0_ref[...], preferred_element_type=jnp.float32)
    _emit_g(g_ref, alad_ref, g, as_ref, ad_ref)


def _post_layer(h2p_ref, den_ref, hin_ref, bi_ref, lnw_ref, lnb_ref):
    num = h2p_ref[0] + h2p_ref[1]
    h2 = num / (den_ref[...] + 1e-16) + bi_ref[...]
    mu = jnp.mean(h2, axis=1, keepdims=True)
    zc = h2 - mu
    var = jnp.mean(zc * zc, axis=1, keepdims=True)
    h2n = zc / jnp.sqrt(var + 1e-5) * lnw_ref[...] + lnb_ref[...]
    return jnp.maximum(h2n, 0.0) + hin_ref[...]


def _mid_body(h2p_ref, den_ref, hin_ref, bi_ref, lnw_ref, lnb_ref, Wn_ref,
              as_ref, ad_ref, hout_ref, g_ref, alad_ref):
    hout = _post_layer(h2p_ref, den_ref, hin_ref, bi_ref, lnw_ref, lnb_ref)
    hout_ref[...] = hout
    g = jnp.dot(hout, Wn_ref[...], preferred_element_type=jnp.float32)
    _emit_g(g_ref, alad_ref, g, as_ref, ad_ref)


def _fin_body(h2p_ref, den_ref, hin_ref, bi_ref, lnw_ref, lnb_ref, decW_ref,
              decb_ref, out_ref):
    hout = _post_layer(h2p_ref, den_ref, hin_ref, bi_ref, lnw_ref, lnb_ref)
    logits = jnp.dot(hout, decW_ref[...],
                     preferred_element_type=jnp.float32) + decb_ref[...]
    sg = jax.nn.sigmoid(logits)
    rid = lax.broadcasted_iota(jnp.int32, (RB, 1), 0) + pl.program_id(0) * RB
    sg = jnp.where(rid < N, sg, 0.0)

    @pl.when(pl.program_id(0) == 0)
    def _():
        out_ref[...] = jnp.zeros_like(out_ref)

    out_ref[...] += jnp.sum(sg, axis=0, keepdims=True)


_full = lambda shape: pl.BlockSpec(shape, lambda i: tuple(0 for _ in shape))

_enc_call = pl.pallas_call(
    _enc_body,
    grid=(NBLK,),
    in_specs=[
        pl.BlockSpec((RB, D), lambda i: (i, 0)),
        _full((D, D)), _full((1, D)), _full((D, D)), _full((1, D)),
        _full((1, D)),
    ],
    out_specs=[
        pl.BlockSpec((RB, D), lambda i: (i, 0)),
        pl.BlockSpec((RB, D), lambda i: (i, 0)),
        pl.BlockSpec((1, 1, RB), lambda i: (i, 0, 0)),
    ],
    out_shape=[
        jax.ShapeDtypeStruct((NP, D), jnp.float32),
        jax.ShapeDtypeStruct((NP, D), jnp.float32),
        jax.ShapeDtypeStruct((NBLK, 1, RB), jnp.int32),
    ],
)

_mid_call = pl.pallas_call(
    _mid_body,
    grid=(NBLK,),
    in_specs=[
        pl.BlockSpec((2, RB, D), lambda i: (0, i, 0)),
        pl.BlockSpec((RB, 1), lambda i: (i, 0)),
        pl.BlockSpec((RB, D), lambda i: (i, 0)),
        _full((1, D)), _full((1, D)), _full((1, D)), _full((D, D)),
        _full((1, D)), _full((1, D)),
    ],
    out_specs=[
        pl.BlockSpec((RB, D), lambda i: (i, 0)),
        pl.BlockSpec((RB, D), lambda i: (i, 0)),
        pl.BlockSpec((1, 1, RB), lambda i: (i, 0, 0)),
    ],
    out_shape=[
        jax.ShapeDtypeStruct((NP, D), jnp.float32),
        jax.ShapeDtypeStruct((NP, D), jnp.float32),
        jax.ShapeDtypeStruct((NBLK, 1, RB), jnp.int32),
    ],
)

_fin_call = pl.pallas_call(
    _fin_body,
    grid=(NBLK,),
    in_specs=[
        pl.BlockSpec((2, RB, D), lambda i: (0, i, 0)),
        pl.BlockSpec((RB, 1), lambda i: (i, 0)),
        pl.BlockSpec((RB, D), lambda i: (i, 0)),
        _full((1, D)), _full((1, D)), _full((1, D)), _full((D, D)),
        _full((1, D)),
    ],
    out_specs=pl.BlockSpec((1, D), lambda i: (0, 0)),
    out_shape=jax.ShapeDtypeStruct((1, D), jnp.float32),
)


def _sc_body(g_hbm, alad_hbm, src_hbm, dst_hbm, h2p_hbm, den_hbm,
             tab_v, src_c, dst_c, ex_c, vbuf, rows_v, h2_sh, den_sh,
             gsem, ssem):
    c = lax.axis_index("c")
    s = lax.axis_index("s")
    wid = s * 2 + c
    row0 = s * RPT

    zv = jnp.zeros((16,), jnp.float32)

    def _z(i, carry):
        for k in range(D // 16):
            rows_v[0, i, pl.ds(k * 16, 16)] = zv
        return carry

    lax.fori_loop(0, BATCH, _z, 0)
    for k in range(RPT // BATCH):
        pltpu.sync_copy(rows_v.at[0],
                        h2_sh.at[pl.ds(row0 + k * BATCH, BATCH)])
    rem = RPT % BATCH
    pltpu.sync_copy(rows_v.at[0, pl.ds(0, rem)],
                    h2_sh.at[pl.ds(row0 + RPT - rem, rem)])
    for k in range(D // 16):
        vbuf[pl.ds(k * 16, 16)] = zv
    for k in range(RPT // D):
        pltpu.sync_copy(vbuf, den_sh.at[pl.ds(row0 + k * D, D)])

    for k in range(NBLK):
        pltpu.sync_copy(alad_hbm.at[k, 0], tab_v.at[pl.ds(k * RB, RB)])
    neg = jnp.full((16,), -8388737, jnp.int32)
    for k in range((NP - N) // 16):
        tab_v[pl.ds(N + k * 16, 16)] = neg
    plsc.subcore_barrier()

    pltpu.sync_copy(src_hbm.at[wid, pl.ds(0, CH)], src_c.at[0])
    pltpu.sync_copy(dst_hbm.at[wid, pl.ds(0, CH)], dst_c.at[0])
    pltpu.async_copy(g_hbm.at[src_c.at[0, 0]], rows_v.at[0], gsem)

    himask = jnp.int32(-65536)

    def _batch(bi, carry):
        rb = lax.rem(bi, 3)
        nrb = lax.rem(bi + 1, 3)
        drb = lax.rem(bi + 1, 3)
        lbi = lax.rem(bi, CH)
        qb = lax.rem(lax.div(bi, CH), 2)
        nlbi = lax.rem(bi + 1, CH)
        nqb = lax.rem(lax.div(bi + 1, CH), 2)

        @pl.when(bi >= 2)
        def _():
            pltpu.make_async_copy(rows_v.at[drb],
                                  h2_sh.at[dst_c.at[0, 0]], ssem).wait()
            pltpu.make_async_copy(ex_c.at[drb],
                                  den_sh.at[dst_c.at[0, 0]], ssem).wait()

        @pl.when((bi + 1 < NBATCH) & (nlbi == 0))
        def _():
            pltpu.sync_copy(src_hbm.at[wid, pl.ds(bi + 1, CH)],
                            src_c.at[nqb])
            pltpu.sync_copy(dst_hbm.at[wid, pl.ds(bi + 1, CH)],
                            dst_c.at[nqb])

        @pl.when(bi + 1 < NBATCH)
        def _():
            pltpu.async_copy(g_hbm.at[src_c.at[nqb, nlbi]], rows_v.at[nrb],
                             gsem)

        def _work(exbuf, rbuf):
            for j in range(BATCH // 16):
                sv = src_c[qb, lbi, pl.ds(j * 16, 16)]
                dv = dst_c[qb, lbi, pl.ds(j * 16, 16)]
                v1 = plsc.load_gather(tab_v, [sv])
                v2 = plsc.load_gather(tab_v, [dv])
                al = lax.bitcast_convert_type(v1 & himask, jnp.float32)
                ad = lax.bitcast_convert_type(lax.shift_left(v2, 16),
                                              jnp.float32)
                t = al + ad
                exbuf[pl.ds(j * 16, 16)] = jnp.exp(jnp.maximum(t, 0.2 * t))

            pltpu.make_async_copy(g_hbm.at[src_c.at[qb, lbi]], rbuf,
                                  gsem).wait()

            def _scale(g16, inner):
                ex = exbuf[pl.ds(g16 * 16, 16)]
                for kk in range(16):
                    sc = ex[kk]
                    row = g16 * 16 + kk
                    for k in range(D // 16):
                        sl = pl.ds(k * 16, 16)
                        rbuf[row, sl] = rbuf[row, sl] * sc
                return inner

            lax.fori_loop(0, BATCH // 16, _scale, 0)
            pltpu.async_copy(rbuf, h2_sh.at[dst_c.at[qb, lbi]], ssem,
                             add=True)
            pltpu.async_copy(exbuf, den_sh.at[dst_c.at[qb, lbi]], ssem,
                             add=True)

        @pl.when(rb == 0)
        def _():
            _work(ex_c.at[0], rows_v.at[0])

        @pl.when(rb == 1)
        def _():
            _work(ex_c.at[1], rows_v.at[1])

        @pl.when(rb == 2)
        def _():
            _work(ex_c.at[2], rows_v.at[2])

        return carry

    lax.fori_loop(0, NBATCH, _batch, 0)
    for _ in range(2):
        pltpu.make_async_copy(rows_v.at[0], h2_sh.at[dst_c.at[0, 0]],
                              ssem).wait()
        pltpu.make_async_copy(ex_c.at[0], den_sh.at[dst_c.at[0, 0]],
                              ssem).wait()

    plsc.subcore_barrier()
    pltpu.sync_copy(h2_sh.at[pl.ds(row0, RPT)],
                    h2p_hbm.at[c, pl.ds(row0, RPT)])
    for k in range(RPT // D):
        pltpu.sync_copy(den_sh.at[pl.ds(row0 + k * D, D)], vbuf)
        pltpu.sync_copy(vbuf, den_hbm.at[c, s * (RPT // D) + k])


_sc_edge = pl.kernel(
    _sc_body,
    out_type=[
        jax.ShapeDtypeStruct((2, NP, D), jnp.float32),
        jax.ShapeDtypeStruct((2, DB, D), jnp.float32),
    ],
    mesh=plsc.VectorSubcoreMesh(core_axis_name="c", subcore_axis_name="s"),
    scratch_types=[
        pltpu.VMEM((NP,), jnp.int32),
        pltpu.VMEM((2, CH, BATCH), jnp.int32),
        pltpu.VMEM((2, CH, BATCH), jnp.int32),
        pltpu.VMEM((3, BATCH), jnp.float32),
        pltpu.VMEM((D,), jnp.float32),
        pltpu.VMEM((3, BATCH, D), jnp.float32),
        pltpu.VMEM_SHARED((NP, D), jnp.float32),
        pltpu.VMEM_SHARED((NP,), jnp.float32),
        pltpu.SemaphoreType.DMA,
        pltpu.SemaphoreType.DMA,
    ],
    compiler_params=pltpu.CompilerParams(needs_layout_passes=False,
                                         use_tc_tiling_on_sc=False),
)


def _impl(x, edge_index, batch, enc_W, enc_b, W, a_src, a_dst, b, ln_w, ln_b,
          dec_W, dec_b):
    pad = N + (jnp.arange(EPAD - E - N, dtype=jnp.int32) % (NP - N))
    loops = jnp.arange(N, dtype=jnp.int32)
    src = jnp.concatenate([edge_index[0].astype(jnp.int32), loops, pad])
    dst = jnp.concatenate([edge_index[1].astype(jnp.int32), loops, pad])
    src = src.reshape(NT, NBATCH, BATCH)
    dst = dst.reshape(NT, NBATCH, BATCH)

    xp = jnp.zeros((NP, D), jnp.float32).at[:N].set(x)
    r1 = lambda v: v.reshape(1, D)

    h0, g0, alad0 = _enc_call(xp, enc_W, r1(enc_b), W[0], r1(a_src[0]),
                              r1(a_dst[0]))
    h2p0, den0 = _sc_edge(g0, alad0, src, dst)
    dn0 = (den0[0] + den0[1]).reshape(NP, 1)
    h1, g1, alad1 = _mid_call(h2p0, dn0, h0, r1(b[0]), r1(ln_w[0]),
                              r1(ln_b[0]), W[1], r1(a_src[1]), r1(a_dst[1]))
    h2p1, den1 = _sc_edge(g1, alad1, src, dst)
    dn1 = (den1[0] + den1[1]).reshape(NP, 1)
    out = _fin_call(h2p1, dn1, h1, r1(b[1]), r1(ln_w[1]), r1(ln_b[1]),
                    dec_W, r1(dec_b))
    return out.reshape(D)


kernel = jax.jit(_impl)


# speedup vs baseline: 3.5880x; 1.2164x over previous
import jax
import jax.numpy as jnp
from jax import lax
from jax.experimental import pallas as pl
from jax.experimental.pallas import tpu as pltpu
from jax.experimental.pallas import tpu_sc as plsc

N = 10000
D = 128
E = 320000

NP = 10240
RB = 1024
NBLK = NP // RB
NT = 32
BATCH = 96
NBATCH = 108
CH = 2
EPT = NBATCH * BATCH
EPAD = NT * EPT
RPT = NP // 16
DB = NP // 128


def _emit_g(g_ref, alad_ref, g, as_ref, ad_ref):
    g_ref[...] = g
    al = jnp.sum(g * as_ref[...], axis=1)
    ad = jnp.sum(g * ad_ref[...], axis=1)
    ai = lax.bitcast_convert_type(al, jnp.int32) + 0x8000
    di = lax.bitcast_convert_type(ad, jnp.int32) + 0x8000
    alad_ref[0, 0, :] = (ai & jnp.int32(-65536)) | lax.shift_right_logical(
        di, 16)


def _enc_body(x_ref, encW_ref, encb_ref, W0_ref, as_ref, ad_ref,
              h_ref, g_ref, alad_ref):
    h = jnp.dot(x_ref[...], encW_ref[...],
                preferred_element_type=jnp.float32) + encb_ref[...]
    h_ref[...] = h
    g = jnp.dot(h, W0_ref[...], preferred_element_type=jnp.float32)
    _emit_g(g_ref, alad_ref, g, as_ref, ad_ref)


def _post_layer(h2p_ref, den_ref, hin_ref, bi_ref, lnw_ref, lnb_ref):
    num = h2p_ref[0] + h2p_ref[1]
    h2 = num / (den_ref[...] + 1e-16) + bi_ref[...]
    mu = jnp.mean(h2, axis=1, keepdims=True)
    zc = h2 - mu
    var = jnp.mean(zc * zc, axis=1, keepdims=True)
    h2n = zc / jnp.sqrt(var + 1e-5) * lnw_ref[...] + lnb_ref[...]
    return jnp.maximum(h2n, 0.0) + hin_ref[...]


def _mid_body(h2p_ref, den_ref, hin_ref, bi_ref, lnw_ref, lnb_ref, Wn_ref,
              as_ref, ad_ref, hout_ref, g_ref, alad_ref):
    hout = _post_layer(h2p_ref, den_ref, hin_ref, bi_ref, lnw_ref, lnb_ref)
    hout_ref[...] = hout
    g = jnp.dot(hout, Wn_ref[...], preferred_element_type=jnp.float32)
    _emit_g(g_ref, alad_ref, g, as_ref, ad_ref)


def _fin_body(h2p_ref, den_ref, hin_ref, bi_ref, lnw_ref, lnb_ref, decW_ref,
              decb_ref, out_ref):
    hout = _post_layer(h2p_ref, den_ref, hin_ref, bi_ref, lnw_ref, lnb_ref)
    logits = jnp.dot(hout, decW_ref[...],
                     preferred_element_type=jnp.float32) + decb_ref[...]
    sg = jax.nn.sigmoid(logits)
    rid = lax.broadcasted_iota(jnp.int32, (RB, 1), 0) + pl.program_id(0) * RB
    sg = jnp.where(rid < N, sg, 0.0)

    @pl.when(pl.program_id(0) == 0)
    def _():
        out_ref[...] = jnp.zeros_like(out_ref)

    out_ref[...] += jnp.sum(sg, axis=0, keepdims=True)


_full = lambda shape: pl.BlockSpec(shape, lambda i: tuple(0 for _ in shape))

_enc_call = pl.pallas_call(
    _enc_body,
    grid=(NBLK,),
    in_specs=[
        pl.BlockSpec((RB, D), lambda i: (i, 0)),
        _full((D, D)), _full((1, D)), _full((D, D)), _full((1, D)),
        _full((1, D)),
    ],
    out_specs=[
        pl.BlockSpec((RB, D), lambda i: (i, 0)),
        pl.BlockSpec((RB, D), lambda i: (i, 0)),
        pl.BlockSpec((1, 1, RB), lambda i: (i, 0, 0)),
    ],
    out_shape=[
        jax.ShapeDtypeStruct((NP, D), jnp.float32),
        jax.ShapeDtypeStruct((NP, D), jnp.float32),
        jax.ShapeDtypeStruct((NBLK, 1, RB), jnp.int32),
    ],
)

_mid_call = pl.pallas_call(
    _mid_body,
    grid=(NBLK,),
    in_specs=[
        pl.BlockSpec((2, RB, D), lambda i: (0, i, 0)),
        pl.BlockSpec((RB, 1), lambda i: (i, 0)),
        pl.BlockSpec((RB, D), lambda i: (i, 0)),
        _full((1, D)), _full((1, D)), _full((1, D)), _full((D, D)),
        _full((1, D)), _full((1, D)),
    ],
    out_specs=[
        pl.BlockSpec((RB, D), lambda i: (i, 0)),
        pl.BlockSpec((RB, D), lambda i: (i, 0)),
        pl.BlockSpec((1, 1, RB), lambda i: (i, 0, 0)),
    ],
    out_shape=[
        jax.ShapeDtypeStruct((NP, D), jnp.float32),
        jax.ShapeDtypeStruct((NP, D), jnp.float32),
        jax.ShapeDtypeStruct((NBLK, 1, RB), jnp.int32),
    ],
)

_fin_call = pl.pallas_call(
    _fin_body,
    grid=(NBLK,),
    in_specs=[
        pl.BlockSpec((2, RB, D), lambda i: (0, i, 0)),
        pl.BlockSpec((RB, 1), lambda i: (i, 0)),
        pl.BlockSpec((RB, D), lambda i: (i, 0)),
        _full((1, D)), _full((1, D)), _full((1, D)), _full((D, D)),
        _full((1, D)),
    ],
    out_specs=pl.BlockSpec((1, D), lambda i: (0, 0)),
    out_shape=jax.ShapeDtypeStruct((1, D), jnp.float32),
)


def _sc_body(g_hbm, alad_hbm, src_hbm, dst_hbm, h2p_hbm, den_hbm,
             tab_v, src_c, dst_c, ex_c, rows_v, h2_sh, den_sh,
             gsem, ssem, isem):
    c = lax.axis_index("c")
    s = lax.axis_index("s")
    wid = s * 2 + c
    row0 = s * RPT

    zv = jnp.zeros((16,), jnp.float32)

    def _z(i, carry):
        for k in range(D // 16):
            rows_v[0, i, pl.ds(k * 16, 16)] = zv
        return carry

    lax.fori_loop(0, BATCH, _z, 0)
    for k in range(RPT // BATCH):
        pltpu.sync_copy(rows_v.at[0],
                        h2_sh.at[pl.ds(row0 + k * BATCH, BATCH)])
    rem = RPT % BATCH
    pltpu.sync_copy(rows_v.at[0, pl.ds(0, rem)],
                    h2_sh.at[pl.ds(row0 + RPT - rem, rem)])
    for k in range(RPT // D):
        pltpu.sync_copy(rows_v.at[0, 0], den_sh.at[pl.ds(row0 + k * D, D)])

    for k in range(NBLK):
        pltpu.sync_copy(alad_hbm.at[k, 0], tab_v.at[pl.ds(k * RB, RB)])
    neg = jnp.full((16,), -8388737, jnp.int32)
    for k in range((NP - N) // 16):
        tab_v[pl.ds(N + k * 16, 16)] = neg
    plsc.subcore_barrier()

    pltpu.sync_copy(src_hbm.at[wid, 0], src_c.at[0])
    pltpu.sync_copy(dst_hbm.at[wid, 0], dst_c.at[0])
    pltpu.async_copy(src_hbm.at[wid, 1], src_c.at[1], isem)
    pltpu.async_copy(dst_hbm.at[wid, 1], dst_c.at[1], isem)
    pltpu.async_copy(g_hbm.at[src_c.at[0]], rows_v.at[0], gsem)

    himask = jnp.int32(-65536)

    def _batch(bi, carry):
        rb = lax.rem(bi, 3)
        nrb = lax.rem(bi + 1, 3)
        drb = lax.rem(bi + 1, 3)
        qb = lax.rem(bi, 4)
        nqb = lax.rem(bi + 1, 4)

        @pl.when(bi >= 2)
        def _():
            pltpu.make_async_copy(rows_v.at[drb],
                                  h2_sh.at[dst_c.at[0]], ssem).wait()
            pltpu.make_async_copy(ex_c.at[drb],
                                  den_sh.at[dst_c.at[0]], ssem).wait()

        @pl.when(bi + 1 < NBATCH)
        def _():
            pltpu.make_async_copy(src_hbm.at[wid, 0],
                                  src_c.at[nqb], isem).wait()
            pltpu.make_async_copy(dst_hbm.at[wid, 0],
                                  dst_c.at[nqb], isem).wait()

            @pl.when(bi + 2 < NBATCH)
            def _():
                fqb = lax.rem(bi + 2, 4)
                pltpu.async_copy(src_hbm.at[wid, bi + 2], src_c.at[fqb],
                                 isem)
                pltpu.async_copy(dst_hbm.at[wid, bi + 2], dst_c.at[fqb],
                                 isem)

            pltpu.async_copy(g_hbm.at[src_c.at[nqb]], rows_v.at[nrb], gsem)

        def _work(exbuf, rbuf):
            for j in range(BATCH // 16):
                sv = src_c[qb, pl.ds(j * 16, 16)]
                dv = dst_c[qb, pl.ds(j * 16, 16)]
                v1 = plsc.load_gather(tab_v, [sv])
                v2 = plsc.load_gather(tab_v, [dv])
                al = lax.bitcast_convert_type(v1 & himask, jnp.float32)
                ad = lax.bitcast_convert_type(lax.shift_left(v2, 16),
                                              jnp.float32)
                t = al + ad
                exbuf[pl.ds(j * 16, 16)] = jnp.exp(jnp.maximum(t, 0.2 * t))

            pltpu.make_async_copy(g_hbm.at[src_c.at[qb]], rbuf,
                                  gsem).wait()

            def _scale(g16, inner):
                ex = exbuf[pl.ds(g16 * 16, 16)]
                for kk in range(16):
                    sc = ex[kk]
                    row = g16 * 16 + kk
                    for k in range(D // 16):
                        sl = pl.ds(k * 16, 16)
                        rbuf[row, sl] = rbuf[row, sl] * sc
                return inner

            lax.fori_loop(0, BATCH // 16, _scale, 0)
            pltpu.async_copy(rbuf, h2_sh.at[dst_c.at[qb]], ssem,
                             add=True)
            pltpu.async_copy(exbuf, den_sh.at[dst_c.at[qb]], ssem,
                             add=True)

        @pl.when(rb == 0)
        def _():
            _work(ex_c.at[0], rows_v.at[0])

        @pl.when(rb == 1)
        def _():
            _work(ex_c.at[1], rows_v.at[1])

        @pl.when(rb == 2)
        def _():
            _work(ex_c.at[2], rows_v.at[2])

        return carry

    lax.fori_loop(0, NBATCH, _batch, 0)
    for _ in range(2):
        pltpu.make_async_copy(rows_v.at[0], h2_sh.at[dst_c.at[0]],
                              ssem).wait()
        pltpu.make_async_copy(ex_c.at[0], den_sh.at[dst_c.at[0]],
                              ssem).wait()

    plsc.subcore_barrier()
    pltpu.sync_copy(h2_sh.at[pl.ds(row0, RPT)],
                    h2p_hbm.at[c, pl.ds(row0, RPT)])
    for k in range(RPT // D):
        pltpu.sync_copy(den_sh.at[pl.ds(row0 + k * D, D)], rows_v.at[0, 0])
        pltpu.sync_copy(rows_v.at[0, 0], den_hbm.at[c, s * (RPT // D) + k])


_sc_edge = pl.kernel(
    _sc_body,
    out_type=[
        jax.ShapeDtypeStruct((2, NP, D), jnp.float32),
        jax.ShapeDtypeStruct((2, DB, D), jnp.float32),
    ],
    mesh=plsc.VectorSubcoreMesh(core_axis_name="c", subcore_axis_name="s"),
    scratch_types=[
        pltpu.VMEM((NP,), jnp.int32),
        pltpu.VMEM((4, BATCH), jnp.int32),
        pltpu.VMEM((4, BATCH), jnp.int32),
        pltpu.VMEM((3, BATCH), jnp.float32),
        pltpu.VMEM((3, BATCH, D), jnp.float32),
        pltpu.VMEM_SHARED((NP, D), jnp.float32),
        pltpu.VMEM_SHARED((NP,), jnp.float32),
        pltpu.SemaphoreType.DMA,
        pltpu.SemaphoreType.DMA,
        pltpu.SemaphoreType.DMA,
    ],
    compiler_params=pltpu.CompilerParams(needs_layout_passes=False,
                                         use_tc_tiling_on_sc=False),
)


def _impl(x, edge_index, batch, enc_W, enc_b, W, a_src, a_dst, b, ln_w, ln_b,
          dec_W, dec_b):
    pad = N + (jnp.arange(EPAD - E - N, dtype=jnp.int32) % (NP - N))
    loops = jnp.arange(N, dtype=jnp.int32)
    src = jnp.concatenate([edge_index[0].astype(jnp.int32), loops, pad])
    dst = jnp.concatenate([edge_index[1].astype(jnp.int32), loops, pad])
    src = src.reshape(NT, NBATCH, BATCH)
    dst = dst.reshape(NT, NBATCH, BATCH)

    xp = jnp.zeros((NP, D), jnp.float32).at[:N].set(x)
    r1 = lambda v: v.reshape(1, D)

    h0, g0, alad0 = _enc_call(xp, enc_W, r1(enc_b), W[0], r1(a_src[0]),
                              r1(a_dst[0]))
    h2p0, den0 = _sc_edge(g0, alad0, src, dst)
    dn0 = (den0[0] + den0[1]).reshape(NP, 1)
    h1, g1, alad1 = _mid_call(h2p0, dn0, h0, r1(b[0]), r1(ln_w[0]),
                              r1(ln_b[0]), W[1], r1(a_src[1]), r1(a_dst[1]))
    h2p1, den1 = _sc_edge(g1, alad1, src, dst)
    dn1 = (den1[0] + den1[1]).reshape(NP, 1)
    out = _fin_call(h2p1, dn1, h1, r1(b[1]), r1(ln_w[1]), r1(ln_b[1]),
                    dec_W, r1(dec_b))
    return out.reshape(D)


kernel = jax.jit(_impl)
